# baseline (speedup 1.0000x reference)
import functools

import jax
import jax.numpy as jnp
from jax import lax
from jax.experimental import pallas as pl
from jax.experimental.pallas import tpu as pltpu
from jax.experimental.pallas import tpu_sc as plsc

IDX_W = 128
CHUNK_ROWS = 8 * IDX_W


@functools.partial(jax.jit, static_argnames=("interpret",))
def _sc_embed(idx2d, val_flat, weight, interpret=False):
    n_idx_rows, idx_w = idx2d.shape
    n = n_idx_rows * idx_w
    d = weight.shape[1]
    info = plsc.get_sparse_core_info()
    nc, ns, nl = info.num_cores, info.num_subcores, info.num_lanes
    nw = nc * ns
    per_w = n // nw
    n_chunks = per_w // CHUNK_ROWS
    assert per_w % CHUNK_ROWS == 0 and n % nw == 0 and d == nl
    rows_per_chunk = CHUNK_ROWS // IDX_W

    mesh = plsc.VectorSubcoreMesh(core_axis_name="c", subcore_axis_name="s")

    @functools.partial(
        pl.kernel,
        out_type=jax.ShapeDtypeStruct((n, d), jnp.float32),
        mesh=mesh,
        scratch_types=[
            pltpu.VMEM((rows_per_chunk, IDX_W), jnp.int32),
            pltpu.VMEM((CHUNK_ROWS,), jnp.float32),
            pltpu.VMEM((CHUNK_ROWS, d), jnp.float32),
            pltpu.SemaphoreType.DMA,
        ],
        compiler_params=pltpu.CompilerParams(use_tc_tiling_on_sc=False),
        interpret=interpret,
    )
    def k(idx_hbm, val_hbm, w_hbm, out_hbm, idx_v, val_v, rows_v, sem):
        wid = lax.axis_index("s") * nc + lax.axis_index("c")

        def chunk_body(ci, carry):
            base = pl.multiple_of(wid * per_w + ci * CHUNK_ROWS, CHUNK_ROWS)
            idx_base = pl.multiple_of(base // IDX_W, CHUNK_ROWS // IDX_W)
            pltpu.sync_copy(idx_hbm.at[pl.ds(idx_base, rows_per_chunk)], idx_v)
            pltpu.sync_copy(val_hbm.at[pl.ds(base, CHUNK_ROWS)], val_v)
            copies = [
                pltpu.make_async_copy(
                    w_hbm.at[idx_v.at[r]],
                    rows_v.at[pl.ds(r * IDX_W, IDX_W)],
                    sem,
                )
                for r in range(rows_per_chunk)
            ]
            for c in copies:
                c.start()
            for c in copies:
                c.wait()

            def scale_body(jb, carry2):
                v16 = val_v[pl.ds(jb * nl, nl)]
                for r in range(nl):
                    row = jb * nl + r
                    bc = jnp.full((nl,), v16[r], jnp.float32)
                    rows_v[row] = rows_v[row] * bc
                return carry2

            lax.fori_loop(0, CHUNK_ROWS // nl, scale_body, 0, unroll=False)
            pltpu.sync_copy(rows_v, out_hbm.at[pl.ds(base, CHUNK_ROWS)])
            return carry

        lax.fori_loop(0, n_chunks, chunk_body, 0, unroll=False)

    return k(idx2d, val_flat, weight)


def kernel(feature_idx, feature_value, weight):
    b, f = feature_idx.shape
    d = weight.shape[1]
    idx2d = feature_idx.reshape(-1, IDX_W).astype(jnp.int32)
    val_flat = feature_value.reshape(-1)
    out = _sc_embed(idx2d, val_flat, weight)
    return out.reshape(b, f, d)


# speedup vs baseline: 5.2659x; 5.2659x over previous
import functools

import jax
import jax.numpy as jnp
from jax import lax
from jax.experimental import pallas as pl
from jax.experimental.pallas import tpu as pltpu
from jax.experimental.pallas import tpu_sc as plsc

_LANES = 16


@jax.jit
def _sc_embed_t(idx_t, val_t, w3):
    nf, nb = idx_t.shape
    nc_w, d_per_core, nv = w3.shape
    info = plsc.get_sparse_core_info()
    nc, ns = info.num_cores, info.num_subcores
    assert nc == nc_w
    nd = nc * d_per_core
    b_per_tile = nb // ns
    mesh = plsc.VectorSubcoreMesh(core_axis_name="c", subcore_axis_name="s")

    @functools.partial(
        pl.kernel,
        out_type=jax.ShapeDtypeStruct((nf, nd, nb), jnp.float32),
        mesh=mesh,
        scratch_types=[
            pltpu.VMEM((nf * b_per_tile,), jnp.int32),
            pltpu.VMEM((nf * b_per_tile,), jnp.float32),
            pltpu.VMEM_SHARED((nv,), jnp.float32),
            pltpu.VMEM((b_per_tile,), jnp.float32),
            pltpu.VMEM((b_per_tile,), jnp.float32),
            pltpu.SemaphoreType.DMA,
            pltpu.SemaphoreType.DMA,
        ],
    )
    def k(idx_hbm, val_hbm, w_hbm, out_hbm, idx_v, val_v, plane_s, res0, res1, sem0, sem1):
        cid = lax.axis_index("c")
        sid = lax.axis_index("s")
        b0 = sid * b_per_tile

        for ff in range(nf):
            pltpu.sync_copy(
                idx_hbm.at[ff, pl.ds(b0, b_per_tile)],
                idx_v.at[pl.ds(ff * b_per_tile, b_per_tile)],
            )
            pltpu.sync_copy(
                val_hbm.at[ff, pl.ds(b0, b_per_tile)],
                val_v.at[pl.ds(ff * b_per_tile, b_per_tile)],
            )

        res = (res0, res1)
        sem = (sem0, sem1)

        def gather_start(f, buf):
            off = pl.multiple_of(f * b_per_tile, b_per_tile)
            idx_slice = idx_v.at[pl.ds(off, b_per_tile)]
            return pltpu.make_async_copy(
                plane_s.at[idx_slice], res[buf], sem[buf]
            )

        def scale_and_store(f, buf, d):
            def mul_body(i, carry):
                off = i * _LANES
                voff = pl.multiple_of(f * b_per_tile + off, _LANES)
                res[buf][pl.ds(off, _LANES)] = (
                    res[buf][pl.ds(off, _LANES)] * val_v[pl.ds(voff, _LANES)]
                )
                return carry

            lax.fori_loop(0, b_per_tile // _LANES, mul_body, 0, unroll=8)
            pltpu.sync_copy(res[buf], out_hbm.at[f, d, pl.ds(b0, b_per_tile)])

        for dd in range(d_per_core):
            plsc.subcore_barrier()

            @pl.when(sid == 0)
            def _load_plane():
                pltpu.sync_copy(w_hbm.at[cid, dd], plane_s)

            plsc.subcore_barrier()
            d = cid * d_per_core + dd

            gather_start(0, 0).start()

            def pair_body(j, carry):
                f0 = j * 2
                f1 = f0 + 1
                gather_start(f1, 1).start()
                gather_start(f0, 0).wait()
                scale_and_store(f0, 0, d)

                @pl.when(j < nf // 2 - 1)
                def _next():
                    gather_start(f0 + 2, 0).start()

                gather_start(f1, 1).wait()
                scale_and_store(f1, 1, d)
                return carry

            lax.fori_loop(0, nf // 2, pair_body, 0, unroll=False)

    return k(idx_t, val_t, w3)


def kernel(feature_idx, feature_value, weight):
    nv, nd = weight.shape
    idx_t = feature_idx.T.astype(jnp.int32)
    val_t = feature_value.T
    w3 = weight.T.reshape(2, nd // 2, nv)
    out3 = _sc_embed_t(idx_t, val_t, w3)
    return out3.transpose(2, 0, 1)


# speedup vs baseline: 6.5332x; 1.2407x over previous
import functools

import jax
import jax.numpy as jnp
from jax import lax
from jax.experimental import pallas as pl
from jax.experimental.pallas import tpu as pltpu
from jax.experimental.pallas import tpu_sc as plsc

_LANES = 16


@jax.jit
def _sc_embed_t(idx_t, val_t, w3):
    nf, nb = idx_t.shape
    nc_w, d_per_core, nv = w3.shape
    info = plsc.get_sparse_core_info()
    nc, ns = info.num_cores, info.num_subcores
    assert nc == nc_w
    nd = nc * d_per_core
    b_per_tile = nb // ns
    mesh = plsc.VectorSubcoreMesh(core_axis_name="c", subcore_axis_name="s")

    @functools.partial(
        pl.kernel,
        out_type=jax.ShapeDtypeStruct((nf, nd, nb), jnp.float32),
        mesh=mesh,
        scratch_types=[
            pltpu.VMEM((nf * b_per_tile,), jnp.int32),
            pltpu.VMEM((nf * b_per_tile,), jnp.float32),
            pltpu.VMEM_SHARED((nv,), jnp.float32),
            pltpu.VMEM((b_per_tile,), jnp.float32),
            pltpu.VMEM((b_per_tile,), jnp.float32),
            pltpu.VMEM((b_per_tile,), jnp.float32),
            pltpu.VMEM((b_per_tile,), jnp.float32),
            pltpu.SemaphoreType.DMA,
            pltpu.SemaphoreType.DMA,
            pltpu.SemaphoreType.DMA,
            pltpu.SemaphoreType.DMA,
            pltpu.SemaphoreType.DMA,
            pltpu.SemaphoreType.DMA,
        ],
    )
    def k(idx_hbm, val_hbm, w_hbm, out_hbm, idx_v, val_v, ps0,
          res0, res1, ob0, ob1, gsem0, gsem1, osem0, osem1, ssem, psem):
        cid = lax.axis_index("c")
        sid = lax.axis_index("s")
        b0 = sid * b_per_tile
        res = (res0, res1)
        ob = (ob0, ob1)
        gsem = (gsem0, gsem1)
        osem = (osem0, osem1)

        @pl.when(sid == 0)
        def _start_plane0():
            pltpu.make_async_copy(w_hbm.at[cid, 0], ps0, psem).start()

        stage = []
        for ff in range(nf):
            stage.append(pltpu.make_async_copy(
                idx_hbm.at[ff, pl.ds(b0, b_per_tile)],
                idx_v.at[pl.ds(ff * b_per_tile, b_per_tile)],
                ssem,
            ))
            stage.append(pltpu.make_async_copy(
                val_hbm.at[ff, pl.ds(b0, b_per_tile)],
                val_v.at[pl.ds(ff * b_per_tile, b_per_tile)],
                ssem,
            ))
        for c in stage:
            c.start()
        for c in stage:
            c.wait()

        @pl.when(sid == 0)
        def _wait_plane0():
            pltpu.make_async_copy(w_hbm.at[cid, 0], ps0, psem).wait()

        plsc.subcore_barrier()

        def gather_start(f, buf):
            off = pl.multiple_of(f * b_per_tile, b_per_tile)
            idx_slice = idx_v.at[pl.ds(off, b_per_tile)]
            return pltpu.make_async_copy(
                ps0.at[idx_slice], res[buf], gsem[buf]
            )

        def out_write(f, buf, d):
            return pltpu.make_async_copy(
                ob[buf], out_hbm.at[f, d, pl.ds(b0, b_per_tile)], osem[buf]
            )

        def process(f, buf, d, j):
            gather_start(f, buf).wait()

            @pl.when(j > 0)
            def _drain_prev_write():
                out_write(f, buf, d).wait()

            def mul_body(i, carry):
                off = i * _LANES
                voff = pl.multiple_of(f * b_per_tile + off, _LANES)
                ob[buf][pl.ds(off, _LANES)] = (
                    res[buf][pl.ds(off, _LANES)] * val_v[pl.ds(voff, _LANES)]
                )
                return carry

            lax.fori_loop(0, b_per_tile // _LANES, mul_body, 0, unroll=8)
            out_write(f, buf, d).start()

        for dd in range(d_per_core):
            d = cid * d_per_core + dd

            gather_start(0, 0).start()
            gather_start(1, 1).start()

            def pair_body(j, carry):
                f0 = j * 2
                f1 = f0 + 1
                process(f0, 0, d, j + dd)

                @pl.when(j < nf // 2 - 1)
                def _g0():
                    gather_start(f0 + 2, 0).start()

                process(f1, 1, d, j + dd)

                @pl.when(j < nf // 2 - 1)
                def _g1():
                    gather_start(f1 + 2, 1).start()

                return carry

            lax.fori_loop(0, nf // 2, pair_body, 0, unroll=False)

            plsc.subcore_barrier()

            if dd + 1 < d_per_core:
                @pl.when(sid == 0)
                def _load_next_plane():
                    c = pltpu.make_async_copy(
                        w_hbm.at[cid, dd + 1], ps0, psem
                    )
                    c.start()
                    c.wait()

                plsc.subcore_barrier()
            else:
                pass

        out_write(nf - 2, 0, nd - 1).wait()
        out_write(nf - 1, 1, nd - 1).wait()

    return k(idx_t, val_t, w3)


def kernel(feature_idx, feature_value, weight):
    nv, nd = weight.shape
    idx_t = feature_idx.T.astype(jnp.int32)
    val_t = feature_value.T
    w3 = weight.T.reshape(2, nd // 2, nv)
    out3 = _sc_embed_t(idx_t, val_t, w3)
    return out3.transpose(2, 0, 1)


# speedup vs baseline: 6.6413x; 1.0165x over previous
import functools

import jax
import jax.numpy as jnp
from jax import lax
from jax.experimental import pallas as pl
from jax.experimental.pallas import tpu as pltpu
from jax.experimental.pallas import tpu_sc as plsc

_LANES = 16


@jax.jit
def _sc_embed_t(idx_t, val_t, w3):
    nf, nb = idx_t.shape
    nc_w, d_per_core, nv = w3.shape
    info = plsc.get_sparse_core_info()
    nc, ns = info.num_cores, info.num_subcores
    assert nc == nc_w
    nd = nc * d_per_core
    b_per_tile = nb // ns
    mesh = plsc.VectorSubcoreMesh(core_axis_name="c", subcore_axis_name="s")

    @functools.partial(
        pl.kernel,
        out_type=jax.ShapeDtypeStruct((nf, nd, nb), jnp.float32),
        mesh=mesh,
        scratch_types=[
            pltpu.VMEM((nf * b_per_tile,), jnp.int32),
            pltpu.VMEM((nf * b_per_tile,), jnp.float32),
            pltpu.VMEM_SHARED((nv,), jnp.float32),
            pltpu.VMEM((2 * b_per_tile,), jnp.float32),
            pltpu.VMEM((2 * b_per_tile,), jnp.float32),
            pltpu.VMEM((2 * b_per_tile,), jnp.float32),
            pltpu.VMEM((2 * b_per_tile,), jnp.float32),
            pltpu.SemaphoreType.DMA,
            pltpu.SemaphoreType.DMA,
            pltpu.SemaphoreType.DMA,
            pltpu.SemaphoreType.DMA,
            pltpu.SemaphoreType.DMA,
            pltpu.SemaphoreType.DMA,
        ],
    )
    def k(idx_hbm, val_hbm, w_hbm, out_hbm, idx_v, val_v, ps0,
          res0, res1, ob0, ob1, gsem0, gsem1, osem0, osem1, ssem, psem):
        cid = lax.axis_index("c")
        sid = lax.axis_index("s")
        b0 = sid * b_per_tile
        res = (res0, res1)
        ob = (ob0, ob1)
        gsem = (gsem0, gsem1)
        osem = (osem0, osem1)

        @pl.when(sid == 0)
        def _start_plane0():
            pltpu.make_async_copy(w_hbm.at[cid, 0], ps0, psem).start()

        stage = []
        for ff in range(nf):
            stage.append(pltpu.make_async_copy(
                idx_hbm.at[ff, pl.ds(b0, b_per_tile)],
                idx_v.at[pl.ds(ff * b_per_tile, b_per_tile)],
                ssem,
            ))
            stage.append(pltpu.make_async_copy(
                val_hbm.at[ff, pl.ds(b0, b_per_tile)],
                val_v.at[pl.ds(ff * b_per_tile, b_per_tile)],
                ssem,
            ))
        for c in stage:
            c.start()
        for c in stage:
            c.wait()

        @pl.when(sid == 0)
        def _wait_plane0():
            pltpu.make_async_copy(w_hbm.at[cid, 0], ps0, psem).wait()

        plsc.subcore_barrier()

        def gather_start(p, buf):
            off = pl.multiple_of(p * 2 * b_per_tile, 2 * b_per_tile)
            idx_slice = idx_v.at[pl.ds(off, 2 * b_per_tile)]
            return pltpu.make_async_copy(
                ps0.at[idx_slice], res[buf], gsem[buf]
            )

        def out_write(p, buf, d, half):
            return pltpu.make_async_copy(
                ob[buf].at[pl.ds(half * b_per_tile, b_per_tile)],
                out_hbm.at[p * 2 + half, d, pl.ds(b0, b_per_tile)],
                osem[buf],
            )

        def process(p, buf, d, j):
            gather_start(p, buf).wait()

            @pl.when(j > 0)
            def _drain_prev_writes():
                out_write(p, buf, d, 0).wait()
                out_write(p, buf, d, 1).wait()

            def mul_body(i, carry):
                off = i * _LANES
                voff = pl.multiple_of(p * 2 * b_per_tile + off, _LANES)
                ob[buf][pl.ds(off, _LANES)] = (
                    res[buf][pl.ds(off, _LANES)] * val_v[pl.ds(voff, _LANES)]
                )
                return carry

            lax.fori_loop(0, 2 * b_per_tile // _LANES, mul_body, 0, unroll=8)
            out_write(p, buf, d, 0).start()
            out_write(p, buf, d, 1).start()

        for dd in range(d_per_core):
            d = cid * d_per_core + dd

            gather_start(0, 0).start()
            gather_start(1, 1).start()
            n_pairs = nf // 2

            def quad_body(j, carry):
                p0 = j * 2
                p1 = p0 + 1
                process(p0, 0, d, j + dd)

                @pl.when(p0 + 2 < n_pairs)
                def _g0():
                    gather_start(p0 + 2, 0).start()

                process(p1, 1, d, j + dd)

                @pl.when(p1 + 2 < n_pairs)
                def _g1():
                    gather_start(p1 + 2, 1).start()

                return carry

            lax.fori_loop(0, n_pairs // 2, quad_body, 0, unroll=False)
            process(n_pairs - 1, 0, d, 1 + dd)

            plsc.subcore_barrier()

            if dd + 1 < d_per_core:
                @pl.when(sid == 0)
                def _load_next_plane():
                    c = pltpu.make_async_copy(
                        w_hbm.at[cid, dd + 1], ps0, psem
                    )
                    c.start()
                    c.wait()

                plsc.subcore_barrier()
            else:
                pass

        out_write(nf // 2 - 1, 0, nd - 1, 0).wait()
        out_write(nf // 2 - 1, 0, nd - 1, 1).wait()
        out_write(nf // 2 - 2, 1, nd - 1, 0).wait()
        out_write(nf // 2 - 2, 1, nd - 1, 1).wait()

    return k(idx_t, val_t, w3)


def kernel(feature_idx, feature_value, weight):
    nv, nd = weight.shape
    idx_t = feature_idx.T.astype(jnp.int32)
    val_t = feature_value.T
    w3 = weight.T.reshape(2, nd // 2, nv)
    out3 = _sc_embed_t(idx_t, val_t, w3)
    return out3.transpose(2, 0, 1)


# speedup vs baseline: 6.6933x; 1.0078x over previous
import functools

import jax
import jax.numpy as jnp
from jax import lax
from jax.experimental import pallas as pl
from jax.experimental.pallas import tpu as pltpu
from jax.experimental.pallas import tpu_sc as plsc

_LANES = 16


@jax.jit
def _sc_embed_t(idx_t, val_t, w3):
    nf, nb = idx_t.shape
    nc_w, d_per_core, nv = w3.shape
    info = plsc.get_sparse_core_info()
    nc, ns = info.num_cores, info.num_subcores
    assert nc == nc_w
    nd = nc * d_per_core
    b_per_tile = nb // ns
    mesh = plsc.VectorSubcoreMesh(core_axis_name="c", subcore_axis_name="s")

    @functools.partial(
        pl.kernel,
        out_type=jax.ShapeDtypeStruct((nf, nd, nb), jnp.float32),
        mesh=mesh,
        scratch_types=[
            pltpu.VMEM((nf * b_per_tile,), jnp.int32),
            pltpu.VMEM((nf * b_per_tile,), jnp.float32),
            pltpu.VMEM_SHARED((nv,), jnp.float32),
            pltpu.VMEM((2 * b_per_tile,), jnp.float32),
            pltpu.VMEM((2 * b_per_tile,), jnp.float32),
            pltpu.VMEM((2 * b_per_tile,), jnp.float32),
            pltpu.VMEM((2 * b_per_tile,), jnp.float32),
            pltpu.SemaphoreType.DMA,
            pltpu.SemaphoreType.DMA,
            pltpu.SemaphoreType.DMA,
            pltpu.SemaphoreType.DMA,
            pltpu.SemaphoreType.DMA,
            pltpu.SemaphoreType.DMA,
        ],
    )
    def k(idx_hbm, val_hbm, w_hbm, out_hbm, idx_v, val_v, ps0,
          res0, res1, ob0, ob1, gsem0, gsem1, osem0, osem1, ssem, psem):
        cid = lax.axis_index("c")
        sid = lax.axis_index("s")
        b0 = sid * b_per_tile
        res = (res0, res1)
        ob = (ob0, ob1)
        gsem = (gsem0, gsem1)
        osem = (osem0, osem1)

        @pl.when(sid == 0)
        def _start_plane0():
            pltpu.make_async_copy(w_hbm.at[cid, 0], ps0, psem).start()

        stage = []
        for ff in range(nf):
            stage.append(pltpu.make_async_copy(
                idx_hbm.at[ff, pl.ds(b0, b_per_tile)],
                idx_v.at[pl.ds(ff * b_per_tile, b_per_tile)],
                ssem,
            ))
            stage.append(pltpu.make_async_copy(
                val_hbm.at[ff, pl.ds(b0, b_per_tile)],
                val_v.at[pl.ds(ff * b_per_tile, b_per_tile)],
                ssem,
            ))
        for c in stage:
            c.start()
        for c in stage:
            c.wait()

        @pl.when(sid == 0)
        def _wait_plane0():
            pltpu.make_async_copy(w_hbm.at[cid, 0], ps0, psem).wait()

        plsc.subcore_barrier()

        def gather_start(p, buf):
            off = pl.multiple_of(p * 2 * b_per_tile, 2 * b_per_tile)
            idx_slice = idx_v.at[pl.ds(off, 2 * b_per_tile)]
            return pltpu.make_async_copy(
                ps0.at[idx_slice], res[buf], gsem[buf]
            )

        def out_write(p, buf, d, half):
            return pltpu.make_async_copy(
                ob[buf].at[pl.ds(half * b_per_tile, b_per_tile)],
                out_hbm.at[p * 2 + half, d, pl.ds(b0, b_per_tile)],
                osem[buf],
            )

        def process(p, buf, d, j, skip_gather_wait=False):
            if not skip_gather_wait:
                gather_start(p, buf).wait()

            @pl.when(j > 0)
            def _drain_prev_writes():
                out_write(p, buf, d, 0).wait()
                out_write(p, buf, d, 1).wait()

            def mul_body(i, carry):
                off = i * _LANES
                voff = pl.multiple_of(p * 2 * b_per_tile + off, _LANES)
                ob[buf][pl.ds(off, _LANES)] = (
                    res[buf][pl.ds(off, _LANES)] * val_v[pl.ds(voff, _LANES)]
                )
                return carry

            lax.fori_loop(0, 2 * b_per_tile // _LANES, mul_body, 0, unroll=8)
            out_write(p, buf, d, 0).start()
            out_write(p, buf, d, 1).start()

        for dd in range(d_per_core):
            d = cid * d_per_core + dd

            gather_start(0, 0).start()
            gather_start(1, 1).start()
            n_pairs = nf // 2

            def quad_body(j, carry):
                p0 = j * 2
                p1 = p0 + 1
                process(p0, 0, d, j + dd)

                @pl.when(p0 + 2 < n_pairs)
                def _g0():
                    gather_start(p0 + 2, 0).start()

                process(p1, 1, d, j + dd)

                @pl.when(p1 + 2 < n_pairs)
                def _g1():
                    gather_start(p1 + 2, 1).start()

                return carry

            lax.fori_loop(0, n_pairs // 2, quad_body, 0, unroll=False)
            gather_start(n_pairs - 1, 0).wait()

            plsc.subcore_barrier()

            if dd + 1 < d_per_core:
                @pl.when(sid == 0)
                def _load_next_plane_start():
                    pltpu.make_async_copy(
                        w_hbm.at[cid, dd + 1], ps0, psem
                    ).start()

            process(n_pairs - 1, 0, d, 1 + dd, skip_gather_wait=True)

            if dd + 1 < d_per_core:
                @pl.when(sid == 0)
                def _load_next_plane_wait():
                    pltpu.make_async_copy(
                        w_hbm.at[cid, dd + 1], ps0, psem
                    ).wait()

                plsc.subcore_barrier()

        out_write(nf // 2 - 1, 0, nd - 1, 0).wait()
        out_write(nf // 2 - 1, 0, nd - 1, 1).wait()
        out_write(nf // 2 - 2, 1, nd - 1, 0).wait()
        out_write(nf // 2 - 2, 1, nd - 1, 1).wait()

    return k(idx_t, val_t, w3)


def kernel(feature_idx, feature_value, weight):
    nv, nd = weight.shape
    idx_t = feature_idx.T.astype(jnp.int32)
    val_t = feature_value.T
    w3 = weight.T.reshape(2, nd // 2, nv)
    out3 = _sc_embed_t(idx_t, val_t, w3)
    return out3.transpose(2, 0, 1)
